# baseline (speedup 1.0000x reference)
import functools

import jax
import jax.numpy as jnp
from jax import lax
from jax.experimental import pallas as pl
from jax.experimental.pallas import tpu as pltpu
from jax.experimental.pallas import tpu_sc as plsc

HIDDEN = 4096
NUM_EXPERTS = 64
TOP_K = 8
N_GROUPS = 8
EPG = NUM_EXPERTS // N_GROUPS
TOPK_GROUPS = 4
ROUTED_SCALING_FACTOR = 2.5

NC = 2
NS = 16
NW = NC * NS
L = 16


def _score_block(x_ref, w_ref, b_ref, s_ref, sb_ref):
    x = x_ref[...]
    w = w_ref[...]
    s = jax.nn.sigmoid(jnp.dot(x, w, preferred_element_type=jnp.float32))
    s_ref[...] = s
    sb_ref[...] = s + b_ref[...]


def _scores_tc(x_TD, kernel_DE, bias_E, tb=512):
    t = x_TD.shape[0]
    bias_2d = jnp.reshape(bias_E, (1, NUM_EXPERTS)).astype(jnp.float32)
    return pl.pallas_call(
        _score_block,
        grid=(t // tb,),
        in_specs=[
            pl.BlockSpec((tb, HIDDEN), lambda i: (i, 0)),
            pl.BlockSpec((HIDDEN, NUM_EXPERTS), lambda i: (0, 0)),
            pl.BlockSpec((1, NUM_EXPERTS), lambda i: (0, 0)),
        ],
        out_specs=[
            pl.BlockSpec((tb, NUM_EXPERTS), lambda i: (i, 0)),
            pl.BlockSpec((tb, NUM_EXPERTS), lambda i: (i, 0)),
        ],
        out_shape=[
            jax.ShapeDtypeStruct((t, NUM_EXPERTS), jnp.float32),
            jax.ShapeDtypeStruct((t, NUM_EXPERTS), jnp.float32),
        ],
    )(x_TD, kernel_DE, bias_2d)


def _sc_router_body(s_hbm, sb_hbm, wout_hbm, iout_hbm,
                    s_chunk, sb_chunk, wv, iv):
    wid = lax.axis_index("s") * NC + lax.axis_index("c")
    t_total = s_hbm.shape[0]
    tw = t_total // NW
    nb = tw // L
    iota = lax.iota(jnp.int32, L)
    neg = jnp.full((L,), -1e30, jnp.float32)
    zero = jnp.full((L,), 0.0, jnp.float32)
    one = jnp.full((L,), 1, jnp.int32)

    def body(b, carry):
        t0 = wid * tw + b * L
        pltpu.sync_copy(s_hbm.at[pl.ds(t0, L)], s_chunk)
        pltpu.sync_copy(sb_hbm.at[pl.ds(t0, L)], sb_chunk)

        vals = [None] * NUM_EXPERTS
        gs = []
        for g in range(N_GROUPS):
            m1 = m2 = None
            for o in range(EPG):
                e = EPG * g + o
                e_splat = jnp.full((L,), e, jnp.int32)
                sb = plsc.load_gather(sb_chunk, [iota, e_splat])
                vals[e] = sb
                if o == 0:
                    m1, m2 = sb, neg
                else:
                    m2 = jnp.maximum(m2, jnp.minimum(sb, m1))
                    m1 = jnp.maximum(m1, sb)
            gs.append(m1 + m2)

        rank = [jnp.zeros((L,), jnp.int32) for _ in range(N_GROUPS)]
        for g in range(N_GROUPS):
            for h in range(g + 1, N_GROUPS):
                c = (gs[h] > gs[g]).astype(jnp.int32)
                rank[g] = rank[g] + c
                rank[h] = rank[h] + (one - c)
        sel = [rank[g] < TOPK_GROUPS for g in range(N_GROUPS)]

        for e in range(NUM_EXPERTS):
            vals[e] = jnp.where(sel[e // EPG], vals[e], zero)

        wcols = []
        icols = []
        for j in range(TOP_K):
            m = neg
            mi = jnp.zeros((L,), jnp.int32)
            for e in range(NUM_EXPERTS):
                v = vals[e]
                c = v > m
                m = jnp.where(c, v, m)
                mi = jnp.where(c, jnp.full((L,), e, jnp.int32), mi)
            wcols.append(plsc.load_gather(s_chunk, [iota, mi]))
            icols.append(mi)
            if j + 1 < TOP_K:
                for e in range(NUM_EXPERTS):
                    vals[e] = jnp.where(
                        mi == jnp.full((L,), e, jnp.int32), neg, vals[e])

        den = wcols[0]
        for j in range(1, TOP_K):
            den = den + wcols[j]
        den = den + 1e-20
        for j in range(TOP_K):
            j_splat = jnp.full((L,), j, jnp.int32)
            plsc.store_scatter(
                wv, [iota, j_splat], wcols[j] / den * ROUTED_SCALING_FACTOR)
            plsc.store_scatter(iv, [iota, j_splat], icols[j])
        pltpu.sync_copy(wv, wout_hbm.at[pl.ds(t0, L)])
        pltpu.sync_copy(iv, iout_hbm.at[pl.ds(t0, L)])
        return carry

    lax.fori_loop(0, nb, body, 0)


def _make_sc_router(t):
    mesh = plsc.VectorSubcoreMesh(core_axis_name="c", subcore_axis_name="s")
    return pl.kernel(
        _sc_router_body,
        out_type=[
            jax.ShapeDtypeStruct((t, TOP_K), jnp.float32),
            jax.ShapeDtypeStruct((t, TOP_K), jnp.int32),
        ],
        mesh=mesh,
        compiler_params=pltpu.CompilerParams(needs_layout_passes=False),
        scratch_types=[
            pltpu.VMEM((L, NUM_EXPERTS), jnp.float32),
            pltpu.VMEM((L, NUM_EXPERTS), jnp.float32),
            pltpu.VMEM((L, TOP_K), jnp.float32),
            pltpu.VMEM((L, TOP_K), jnp.int32),
        ],
    )


@functools.partial(jax.jit, static_argnames=())
def kernel(x_TD, kernel_DE, bias_E):
    x_TD = jnp.asarray(x_TD, jnp.float32)
    t = x_TD.shape[0]
    s_TE, sb_TE = _scores_tc(x_TD, kernel_DE, bias_E)
    router = _make_sc_router(t)
    weights, indices = router(s_TE, sb_TE)
    return weights, indices


# speedup vs baseline: 1.4997x; 1.4997x over previous
import functools

import jax
import jax.numpy as jnp
from jax import lax
from jax.experimental import pallas as pl
from jax.experimental.pallas import tpu as pltpu
from jax.experimental.pallas import tpu_sc as plsc

HIDDEN = 4096
NUM_EXPERTS = 64
TOP_K = 8
N_GROUPS = 8
EPG = NUM_EXPERTS // N_GROUPS
TOPK_GROUPS = 4
ROUTED_SCALING_FACTOR = 2.5

NC = 2
NS = 16
NW = NC * NS
L = 16
CB = 128


def _score_block(x_ref, w_ref, b_ref, sb_ref):
    x = x_ref[...]
    w = w_ref[...]
    s = jax.nn.sigmoid(jnp.dot(x, w, preferred_element_type=jnp.float32))
    sb_ref[...] = s + b_ref[...]


def _scores_tc(x_TD, kernel_DE, bias_E, tb=512):
    t = x_TD.shape[0]
    bias_2d = jnp.reshape(bias_E, (1, NUM_EXPERTS)).astype(jnp.float32)
    return pl.pallas_call(
        _score_block,
        grid=(t // tb,),
        in_specs=[
            pl.BlockSpec((tb, HIDDEN), lambda i: (i, 0)),
            pl.BlockSpec((HIDDEN, NUM_EXPERTS), lambda i: (0, 0)),
            pl.BlockSpec((1, NUM_EXPERTS), lambda i: (0, 0)),
        ],
        out_specs=pl.BlockSpec((tb, NUM_EXPERTS), lambda i: (i, 0)),
        out_shape=jax.ShapeDtypeStruct((t, NUM_EXPERTS), jnp.float32),
    )(x_TD, kernel_DE, bias_2d)


def _sc_router_body(sb_hbm, b2_hbm, wout_hbm, iout_hbm,
                    sb_chunk, ms_ref, bias_v, wv, iv):
    wid = lax.axis_index("s") * NC + lax.axis_index("c")
    t_total = sb_hbm.shape[0]
    tw = t_total // NW
    nst = tw // CB
    nsb = CB // L
    pltpu.sync_copy(b2_hbm, bias_v)
    iota = lax.iota(jnp.int32, L)
    zero16 = jnp.zeros((L,), jnp.int32)
    neg = jnp.full((L,), -1e30, jnp.float32)
    zero = jnp.full((L,), 0.0, jnp.float32)
    one = jnp.full((L,), 1, jnp.int32)
    esplat = [jnp.full((L,), e, jnp.int32) for e in range(NUM_EXPERTS)]

    def stage_body(st, carry0):
        t0 = wid * tw + st * CB
        pltpu.sync_copy(sb_hbm.at[pl.ds(t0, CB)], sb_chunk)

        def sub_body(i, carry):
            row = iota + i * L

            gs = []
            for g in range(N_GROUPS):
                m1 = m2 = None
                for o in range(EPG):
                    e = EPG * g + o
                    sb = plsc.load_gather(sb_chunk, [row, esplat[e]])
                    ms_ref[e] = sb
                    if o == 0:
                        m1, m2 = sb, neg
                    else:
                        m2 = jnp.maximum(m2, jnp.minimum(sb, m1))
                        m1 = jnp.maximum(m1, sb)
                gs.append(m1 + m2)

            rank = [jnp.zeros((L,), jnp.int32) for _ in range(N_GROUPS)]
            for g in range(N_GROUPS):
                for h in range(g + 1, N_GROUPS):
                    c = (gs[h] > gs[g]).astype(jnp.int32)
                    rank[g] = rank[g] + c
                    rank[h] = rank[h] + (one - c)
            sel = [rank[g] < TOPK_GROUPS for g in range(N_GROUPS)]

            for e in range(NUM_EXPERTS):
                ms_ref[e] = jnp.where(sel[e // EPG], ms_ref[e], zero)

            wcols, icols = [], []
            for j in range(TOP_K):
                vcur = [ms_ref[e] for e in range(NUM_EXPERTS)]
                icur = list(esplat)
                n = NUM_EXPERTS
                while n > 1:
                    nv, ni = [], []
                    for k in range(0, n, 2):
                        c = vcur[k + 1] > vcur[k]
                        nv.append(jnp.where(c, vcur[k + 1], vcur[k]))
                        ni.append(jnp.where(c, icur[k + 1], icur[k]))
                    vcur, icur = nv, ni
                    n //= 2
                m, mi = vcur[0], icur[0]
                be = plsc.load_gather(bias_v, [zero16, mi])
                wcols.append(m - be)
                icols.append(mi)
                if j + 1 < TOP_K:
                    plsc.store_scatter(ms_ref, [mi, iota], neg)

            den = wcols[0]
            for j in range(1, TOP_K):
                den = den + wcols[j]
            den = den + 1e-20
            for j in range(TOP_K):
                plsc.store_scatter(
                    wv, [row, esplat[j]],
                    wcols[j] / den * ROUTED_SCALING_FACTOR)
                plsc.store_scatter(iv, [row, esplat[j]], icols[j])
            return carry

        lax.fori_loop(0, nsb, sub_body, 0)

        pltpu.sync_copy(wv, wout_hbm.at[pl.ds(t0, CB)])
        pltpu.sync_copy(iv, iout_hbm.at[pl.ds(t0, CB)])
        return carry0

    lax.fori_loop(0, nst, stage_body, 0)


def _make_sc_router(t):
    mesh = plsc.VectorSubcoreMesh(core_axis_name="c", subcore_axis_name="s")
    return pl.kernel(
        _sc_router_body,
        out_type=[
            jax.ShapeDtypeStruct((t, TOP_K), jnp.float32),
            jax.ShapeDtypeStruct((t, TOP_K), jnp.int32),
        ],
        mesh=mesh,
        compiler_params=pltpu.CompilerParams(needs_layout_passes=False),
        scratch_types=[
            pltpu.VMEM((CB, NUM_EXPERTS), jnp.float32),
            pltpu.VMEM((NUM_EXPERTS, L), jnp.float32),
            pltpu.VMEM((1, NUM_EXPERTS), jnp.float32),
            pltpu.VMEM((CB, TOP_K), jnp.float32),
            pltpu.VMEM((CB, TOP_K), jnp.int32),
        ],
    )


@functools.partial(jax.jit, static_argnames=())
def kernel(x_TD, kernel_DE, bias_E):
    x_TD = jnp.asarray(x_TD, jnp.float32)
    t = x_TD.shape[0]
    sb_TE = _scores_tc(x_TD, kernel_DE, bias_E)
    router = _make_sc_router(t)
    weights, indices = router(
        sb_TE, jnp.reshape(bias_E, (1, NUM_EXPERTS)).astype(jnp.float32))
    return weights, indices
